# baseline (speedup 1.0000x reference)
import functools

import jax
import jax.numpy as jnp
from jax.experimental import pallas as pl
from jax.experimental.pallas import tpu as pltpu
from jax.experimental.pallas import tpu_sc as plsc

F32 = jnp.float32
BF16 = jnp.bfloat16
_EPS = 1e-5
_H = 16
_KMAX = 64


def _ln(x, g, b):
    mu = jnp.mean(x, axis=-1, keepdims=True)
    var = jnp.mean((x - mu) ** 2, axis=-1, keepdims=True)
    return (x - mu) * jax.lax.rsqrt(var + _EPS) * g + b


def _mmt(a, b):
    return jax.lax.dot_general(a, b, (((1,), (1,)), ((), ())),
                               preferred_element_type=F32)


def _mmt_hi(a, b):
    return jax.lax.dot_general(a, b, (((1,), (1,)), ((), ())),
                               preferred_element_type=F32,
                               precision=jax.lax.Precision.HIGHEST)


def _mm(a, b):
    return jax.lax.dot_general(a, b, (((1,), (0,)), ((), ())),
                               preferred_element_type=F32)


def _summary_body(h_ref, g_ref, b_ref, out_ref, *, inv_t):
    t = pl.program_id(1)
    xn = _ln(h_ref[0], g_ref[...], b_ref[...])

    @pl.when(t == 0)
    def _():
        out_ref[...] = jnp.zeros_like(out_ref)

    out_ref[...] += jnp.sum(xn, axis=0, keepdims=True)[None]

    @pl.when(t == pl.num_programs(1) - 1)
    def _():
        out_ref[...] *= inv_t


def _select_body(sum_ref, wf_ref, bf_ref, act_ref, aw_ref, mem_ref, idx_ref,
                 sel_ref, fq_ref, *, k_sel):
    i = pl.program_id(0)
    ni = pl.num_programs(0)

    @pl.when(i == 0)
    def _():
        fq_ref[...] = _mmt_hi(sum_ref[...], wf_ref[...]) + bf_ref[...]

    fq = fq_ref[...]
    mem = mem_ref[...]
    mh = mem.astype(BF16)
    ml = (mem - mh.astype(F32)).astype(BF16)
    fh = fq.astype(BF16)
    fl = (fq - fh.astype(F32)).astype(BF16)
    rel = _mmt(fh, mh) + (_mmt(fh, ml) + _mmt(fl, mh))
    sel_ref[:, pl.ds(i, 1), :] = (
        rel + aw_ref[0, 0] * act_ref[0])[:, None, :]

    @pl.when(i == ni - 1)
    def _():
        bsz, _, nb = sel_ref.shape
        iota_nb = jax.lax.broadcasted_iota(jnp.int32, (bsz, ni, nb), 2)
        iota_i = jax.lax.broadcasted_iota(jnp.int32, (bsz, ni, nb), 1)
        gidx = iota_i * nb + iota_nb
        iota_k = jax.lax.broadcasted_iota(jnp.int32, (bsz, k_sel), 1)
        big = jnp.int32(ni * nb)

        def body(kk, idxacc):
            sel = sel_ref[...]
            m = jnp.max(jnp.max(sel, axis=2), axis=1)[:, None, None]
            cand = jnp.where(sel >= m, gidx, big)
            j = jnp.min(jnp.min(cand, axis=2), axis=1)[:, None, None]
            sel_ref[...] = jnp.where(gidx == j, -jnp.inf, sel)
            return jnp.where(iota_k == kk, j[:, :, 0], idxacc)

        idx_ref[...] = jax.lax.fori_loop(
            0, k_sel, body, jnp.zeros((bsz, k_sel), jnp.int32))


def _sc_gather(memory, idx_flat):
    bk = idx_flat.shape[0]
    d = memory.shape[1]
    info = plsc.get_sparse_core_info()
    nc = info.num_cores
    nw = nc * info.num_subcores
    bpw = bk // nw
    mesh = plsc.VectorSubcoreMesh(core_axis_name="c", subcore_axis_name="s")

    @functools.partial(
        pl.kernel, mesh=mesh,
        out_type=jax.ShapeDtypeStruct((bk, d), F32),
        scratch_types=[pltpu.VMEM((bpw,), jnp.int32),
                       pltpu.VMEM((bpw, d), F32),
                       pltpu.SemaphoreType.DMA])
    def gk(mem_hbm, idx_hbm, out_hbm, idx_v, rows_v, sem):
        wid = jax.lax.axis_index("s") * nc + jax.lax.axis_index("c")
        base = wid * bpw
        pltpu.sync_copy(idx_hbm.at[pl.ds(base, bpw)], idx_v)
        pltpu.async_copy(mem_hbm.at[idx_v], rows_v, sem).wait()
        pltpu.sync_copy(rows_v, out_hbm.at[pl.ds(base, bpw)])

    return gk(memory, idx_flat)


def _kv_pack_body(tm_ref, wk_ref, bk_ref, wv_ref, bv_ref, mask_ref,
                  km_ref, vm_ref, *, bsz, heads):
    tm = tm_ref[...].astype(BF16)
    kf = _mmt(tm, wk_ref[...]) + bk_ref[...]
    vf = _mmt(tm, wv_ref[...]) + bv_ref[...]
    k_sel = tm_ref.shape[0] // bsz
    d = tm_ref.shape[1]
    mask = mask_ref[...]
    for b in range(bsz):
        kb = kf[b * k_sel:(b + 1) * k_sel].astype(BF16)
        vb = vf[b * k_sel:(b + 1) * k_sel].astype(BF16)
        ktile = jnp.broadcast_to(kb[None], (heads, k_sel, d)).reshape(d, d)
        vtile = jnp.broadcast_to(vb[None], (heads, k_sel, d)).reshape(d, d)
        km_ref[b] = ktile * mask
        vm_ref[b] = vtile * mask


def _attn_body(h_ref, km_ref, vm_ref, mask_ref, ones_ref, wq_ref, bq_ref,
               wo_ref, bo_ref, g_ref, b_ref, gl_ref,
               hu_ref, asum_ref, *, scale, heads):
    t = pl.program_id(1)
    x = h_ref[0]
    xn = _ln(x, g_ref[...], b_ref[...])
    q = _mmt(xn.astype(BF16), wq_ref[...]) + bq_ref[...]
    qb = (q * scale).astype(BF16)
    d = x.shape[1]
    k_sel = d // heads
    s_all = _mmt(qb, km_ref[0])
    e = jnp.exp(s_all)
    eb = e.astype(BF16)
    den = _mm(eb, mask_ref[...])
    p = e / den
    pb = p.astype(BF16)
    psum = _mm(ones_ref[...], pb)
    acc = psum[:, 0:k_sel]
    for hh in range(1, heads):
        acc = acc + psum[:, hh * k_sel:(hh + 1) * k_sel]
    o_all = _mm(pb, vm_ref[0])
    out = _mmt(o_all.astype(BF16), wo_ref[...]) + bo_ref[...]
    gate = 1.0 / (1.0 + jnp.exp(-gl_ref[0, 0]))
    hu_ref[0] = x + gate * out

    @pl.when(t == 0)
    def _():
        asum_ref[...] = jnp.zeros_like(asum_ref)

    asum_ref[...] += acc[None]


def _scatter_body(idx_ref, asum_ref, fa_ref, *, inv_ht):
    b = pl.program_id(0)
    n = fa_ref.shape[2]
    k_sel = idx_ref.shape[1]
    iota = jax.lax.broadcasted_iota(jnp.int32, (1, n), 1)
    fa = jnp.zeros((1, n), F32)
    for kk in range(k_sel):
        fa = fa + jnp.where(iota == idx_ref[b, kk],
                            asum_ref[b, kk] * inv_ht, 0.0)
    fa_ref[...] = fa[None]


def kernel(h, memory, activations, Wq, bq, Wk, bk, Wv, bv, Wo, bo,
           ln_g, ln_b, Wf, bf, activation_weight, gate_logit):
    B, T, d = h.shape
    N = memory.shape[0]
    K = min(_KMAX, N)
    H = _H
    TT = min(512, T)
    nT = T // TT

    g2 = ln_g.reshape(1, d)
    b2 = ln_b.reshape(1, d)
    bq2 = bq.reshape(1, d)
    bo2 = bo.reshape(1, d)
    bf2 = bf.reshape(1, d)
    bk2 = bk.reshape(1, d)
    bv2 = bv.reshape(1, d)
    aw2 = activation_weight.reshape(1, 1)
    gl2 = gate_logit.reshape(1, 1)
    wq_b = Wq.astype(BF16)
    wo_b = Wo.astype(BF16)
    wk_b = Wk.astype(BF16)
    wv_b = Wv.astype(BF16)
    ii = jnp.arange(d, dtype=jnp.int32) // (d // H)
    mask_bd = (ii[:, None] == ii[None, :]).astype(BF16)
    ones_tt = jnp.ones((1, TT), BF16)

    summary = pl.pallas_call(
        functools.partial(_summary_body, inv_t=1.0 / T),
        grid=(B, nT),
        in_specs=[
            pl.BlockSpec((1, TT, d), lambda b_, t_: (b_, t_, 0)),
            pl.BlockSpec((1, d), lambda b_, t_: (0, 0)),
            pl.BlockSpec((1, d), lambda b_, t_: (0, 0)),
        ],
        out_specs=pl.BlockSpec((1, 1, d), lambda b_, t_: (b_, 0, 0)),
        out_shape=jax.ShapeDtypeStruct((B, 1, d), F32),
    )(h, g2, b2)
    summary = summary.reshape(B, d)

    NB = 1024
    nI = N // NB
    act3 = activations.reshape(B, nI, NB).transpose(1, 0, 2)
    idx = pl.pallas_call(
        functools.partial(_select_body, k_sel=K),
        grid=(nI,),
        in_specs=[
            pl.BlockSpec((B, d), lambda i: (0, 0)),
            pl.BlockSpec((d, d), lambda i: (0, 0)),
            pl.BlockSpec((1, d), lambda i: (0, 0)),
            pl.BlockSpec((1, B, NB), lambda i: (i, 0, 0)),
            pl.BlockSpec(memory_space=pltpu.SMEM),
            pl.BlockSpec((NB, d), lambda i: (i, 0)),
        ],
        out_specs=pl.BlockSpec((B, K), lambda i: (0, 0)),
        out_shape=jax.ShapeDtypeStruct((B, K), jnp.int32),
        scratch_shapes=[pltpu.VMEM((B, nI, NB), F32),
                        pltpu.VMEM((B, d), F32)],
    )(summary, Wf, bf2, act3, aw2, memory)

    tm = _sc_gather(memory, idx.reshape(B * K))

    km, vm = pl.pallas_call(
        functools.partial(_kv_pack_body, bsz=B, heads=H),
        in_specs=[pl.BlockSpec(memory_space=pltpu.VMEM)] * 6,
        out_specs=[pl.BlockSpec(memory_space=pltpu.VMEM)] * 2,
        out_shape=[jax.ShapeDtypeStruct((B, d, d), BF16)] * 2,
    )(tm, wk_b, bk2, wv_b, bv2, mask_bd)

    hu, asum = pl.pallas_call(
        functools.partial(_attn_body, scale=(d // H) ** -0.5, heads=H),
        grid=(B, nT),
        in_specs=[
            pl.BlockSpec((1, TT, d), lambda b_, t_: (b_, t_, 0)),
            pl.BlockSpec((1, d, d), lambda b_, t_: (b_, 0, 0)),
            pl.BlockSpec((1, d, d), lambda b_, t_: (b_, 0, 0)),
            pl.BlockSpec((d, d), lambda b_, t_: (0, 0)),
            pl.BlockSpec((1, TT), lambda b_, t_: (0, 0)),
            pl.BlockSpec((d, d), lambda b_, t_: (0, 0)),
            pl.BlockSpec((1, d), lambda b_, t_: (0, 0)),
            pl.BlockSpec((d, d), lambda b_, t_: (0, 0)),
            pl.BlockSpec((1, d), lambda b_, t_: (0, 0)),
            pl.BlockSpec((1, d), lambda b_, t_: (0, 0)),
            pl.BlockSpec((1, d), lambda b_, t_: (0, 0)),
            pl.BlockSpec(memory_space=pltpu.SMEM),
        ],
        out_specs=[
            pl.BlockSpec((1, TT, d), lambda b_, t_: (b_, t_, 0)),
            pl.BlockSpec((1, 1, K), lambda b_, t_: (b_, 0, 0)),
        ],
        out_shape=[
            jax.ShapeDtypeStruct((B, T, d), F32),
            jax.ShapeDtypeStruct((B, 1, K), F32),
        ],
    )(h, km, vm, mask_bd, ones_tt, wq_b, bq2, wo_b, bo2, g2, b2, gl2)
    asum = asum.reshape(B, K)

    fa = pl.pallas_call(
        functools.partial(_scatter_body, inv_ht=1.0 / (H * T)),
        grid=(B,),
        in_specs=[
            pl.BlockSpec(memory_space=pltpu.SMEM),
            pl.BlockSpec(memory_space=pltpu.SMEM),
        ],
        out_specs=pl.BlockSpec((1, 1, N), lambda b_: (b_, 0, 0)),
        out_shape=jax.ShapeDtypeStruct((B, 1, N), F32),
    )(idx, asum)

    return hu, fa.reshape(B, N)


# speedup vs baseline: 1.0280x; 1.0280x over previous
import functools

import jax
import jax.numpy as jnp
from jax.experimental import pallas as pl
from jax.experimental.pallas import tpu as pltpu
from jax.experimental.pallas import tpu_sc as plsc

F32 = jnp.float32
BF16 = jnp.bfloat16
_EPS = 1e-5
_H = 16
_KMAX = 64


def _ln(x, g, b):
    mu = jnp.mean(x, axis=-1, keepdims=True)
    var = jnp.mean((x - mu) ** 2, axis=-1, keepdims=True)
    return (x - mu) * jax.lax.rsqrt(var + _EPS) * g + b


def _mmt(a, b):
    return jax.lax.dot_general(a, b, (((1,), (1,)), ((), ())),
                               preferred_element_type=F32)


def _mmt_hi(a, b):
    return jax.lax.dot_general(a, b, (((1,), (1,)), ((), ())),
                               preferred_element_type=F32,
                               precision=jax.lax.Precision.HIGHEST)


def _mm(a, b):
    return jax.lax.dot_general(a, b, (((1,), (0,)), ((), ())),
                               preferred_element_type=F32)


def _summary_body(h_ref, g_ref, b_ref, out_ref, *, inv_t):
    t = pl.program_id(1)
    xn = _ln(h_ref[0], g_ref[...], b_ref[...])

    @pl.when(t == 0)
    def _():
        out_ref[...] = jnp.zeros_like(out_ref)

    out_ref[...] += jnp.sum(xn, axis=0, keepdims=True)[None]

    @pl.when(t == pl.num_programs(1) - 1)
    def _():
        out_ref[...] *= inv_t


def _select_body(sum_ref, wf_ref, bf_ref, act_ref, aw_ref, mem_ref, idx_ref,
                 sel_ref, *, k_sel):
    fq = _mmt_hi(sum_ref[...], wf_ref[...]) + bf_ref[...]
    mem = mem_ref[...]
    mh = mem.astype(BF16)
    ml = (mem - mh.astype(F32)).astype(BF16)
    fh = fq.astype(BF16)
    fl = (fq - fh.astype(F32)).astype(BF16)
    rel = _mmt(fh, mh) + (_mmt(fh, ml) + _mmt(fl, mh))
    sel_ref[...] = rel + aw_ref[0, 0] * act_ref[...]
    bsz, n = sel_ref.shape
    iota_n = jax.lax.broadcasted_iota(jnp.int32, (bsz, n), 1)
    iota_k = jax.lax.broadcasted_iota(jnp.int32, (bsz, k_sel), 1)

    def body(kk, idxacc):
        sel = sel_ref[...]
        m = jnp.max(sel, axis=1, keepdims=True)
        cand = jnp.where(sel >= m, iota_n, jnp.int32(n))
        j = jnp.min(cand, axis=1, keepdims=True)
        sel_ref[...] = jnp.where(iota_n == j, -jnp.inf, sel)
        return jnp.where(iota_k == kk, j, idxacc)

    idx_ref[...] = jax.lax.fori_loop(
        0, k_sel, body, jnp.zeros((bsz, k_sel), jnp.int32))


def _sc_gather(memory, idx_flat):
    bk = idx_flat.shape[0]
    d = memory.shape[1]
    info = plsc.get_sparse_core_info()
    nc = info.num_cores
    nw = nc * info.num_subcores
    bpw = bk // nw
    mesh = plsc.VectorSubcoreMesh(core_axis_name="c", subcore_axis_name="s")

    @functools.partial(
        pl.kernel, mesh=mesh,
        out_type=jax.ShapeDtypeStruct((bk, d), F32),
        scratch_types=[pltpu.VMEM((bpw,), jnp.int32),
                       pltpu.VMEM((bpw, d), F32),
                       pltpu.SemaphoreType.DMA])
    def gk(mem_hbm, idx_hbm, out_hbm, idx_v, rows_v, sem):
        wid = jax.lax.axis_index("s") * nc + jax.lax.axis_index("c")
        base = wid * bpw
        pltpu.sync_copy(idx_hbm.at[pl.ds(base, bpw)], idx_v)
        pltpu.async_copy(mem_hbm.at[idx_v], rows_v, sem).wait()
        pltpu.sync_copy(rows_v, out_hbm.at[pl.ds(base, bpw)])

    return gk(memory, idx_flat)


def _kv_pack_body(tm_ref, wk_ref, bk_ref, wv_ref, bv_ref, mask_ref,
                  km_ref, vm_ref, *, bsz, heads):
    tm = tm_ref[...].astype(BF16)
    kf = _mmt(tm, wk_ref[...]) + bk_ref[...]
    vf = _mmt(tm, wv_ref[...]) + bv_ref[...]
    k_sel = tm_ref.shape[0] // bsz
    d = tm_ref.shape[1]
    mask = mask_ref[...]
    for b in range(bsz):
        kb = kf[b * k_sel:(b + 1) * k_sel].astype(BF16)
        vb = vf[b * k_sel:(b + 1) * k_sel].astype(BF16)
        ktile = jnp.broadcast_to(kb[None], (heads, k_sel, d)).reshape(d, d)
        vtile = jnp.broadcast_to(vb[None], (heads, k_sel, d)).reshape(d, d)
        km_ref[b] = ktile * mask
        vm_ref[b] = vtile * mask


def _attn_body(h_ref, km_ref, vm_ref, mask_ref, ones_ref, wq_ref, bq_ref,
               wo_ref, bo_ref, g_ref, b_ref, gl_ref,
               hu_ref, asum_ref, *, scale, heads):
    t = pl.program_id(1)
    x = h_ref[0]
    xn = _ln(x, g_ref[...], b_ref[...])
    q = _mmt(xn.astype(BF16), wq_ref[...]) + bq_ref[...]
    qb = (q * scale).astype(BF16)
    d = x.shape[1]
    k_sel = d // heads
    s_all = _mmt(qb, km_ref[0])
    e = jnp.exp(s_all)
    eb = e.astype(BF16)
    den = _mm(eb, mask_ref[...])
    p = e / den
    pb = p.astype(BF16)
    psum = _mm(ones_ref[...], pb)
    acc = psum[:, 0:k_sel]
    for hh in range(1, heads):
        acc = acc + psum[:, hh * k_sel:(hh + 1) * k_sel]
    o_all = _mm(pb, vm_ref[0])
    out = _mmt(o_all.astype(BF16), wo_ref[...]) + bo_ref[...]
    gate = 1.0 / (1.0 + jnp.exp(-gl_ref[0, 0]))
    hu_ref[0] = x + gate * out

    @pl.when(t == 0)
    def _():
        asum_ref[...] = jnp.zeros_like(asum_ref)

    asum_ref[...] += acc[None]


def _scatter_body(idx_ref, asum_ref, fa_ref, *, inv_ht):
    b = pl.program_id(0)
    n = fa_ref.shape[2]
    k_sel = idx_ref.shape[1]
    iota = jax.lax.broadcasted_iota(jnp.int32, (1, n), 1)
    fa = jnp.zeros((1, n), F32)
    for kk in range(k_sel):
        fa = fa + jnp.where(iota == idx_ref[b, kk],
                            asum_ref[b, kk] * inv_ht, 0.0)
    fa_ref[...] = fa[None]


def kernel(h, memory, activations, Wq, bq, Wk, bk, Wv, bv, Wo, bo,
           ln_g, ln_b, Wf, bf, activation_weight, gate_logit):
    B, T, d = h.shape
    N = memory.shape[0]
    K = min(_KMAX, N)
    H = _H
    TT = min(512, T)
    nT = T // TT

    g2 = ln_g.reshape(1, d)
    b2 = ln_b.reshape(1, d)
    bq2 = bq.reshape(1, d)
    bo2 = bo.reshape(1, d)
    bf2 = bf.reshape(1, d)
    bk2 = bk.reshape(1, d)
    bv2 = bv.reshape(1, d)
    aw2 = activation_weight.reshape(1, 1)
    gl2 = gate_logit.reshape(1, 1)
    wq_b = Wq.astype(BF16)
    wo_b = Wo.astype(BF16)
    wk_b = Wk.astype(BF16)
    wv_b = Wv.astype(BF16)
    ii = jnp.arange(d, dtype=jnp.int32) // (d // H)
    mask_bd = (ii[:, None] == ii[None, :]).astype(BF16)
    ones_tt = jnp.ones((1, TT), BF16)

    summary = pl.pallas_call(
        functools.partial(_summary_body, inv_t=1.0 / T),
        grid=(B, nT),
        in_specs=[
            pl.BlockSpec((1, TT, d), lambda b_, t_: (b_, t_, 0)),
            pl.BlockSpec((1, d), lambda b_, t_: (0, 0)),
            pl.BlockSpec((1, d), lambda b_, t_: (0, 0)),
        ],
        out_specs=pl.BlockSpec((1, 1, d), lambda b_, t_: (b_, 0, 0)),
        out_shape=jax.ShapeDtypeStruct((B, 1, d), F32),
    )(h, g2, b2)
    summary = summary.reshape(B, d)

    idx = pl.pallas_call(
        functools.partial(_select_body, k_sel=K),
        in_specs=[pl.BlockSpec(memory_space=pltpu.VMEM)] * 4
        + [pl.BlockSpec(memory_space=pltpu.SMEM),
           pl.BlockSpec(memory_space=pltpu.VMEM)],
        out_specs=pl.BlockSpec(memory_space=pltpu.VMEM),
        out_shape=jax.ShapeDtypeStruct((B, K), jnp.int32),
        scratch_shapes=[pltpu.VMEM((B, N), F32)],
    )(summary, Wf, bf2, activations, aw2, memory)

    tm = _sc_gather(memory, idx.reshape(B * K))

    km, vm = pl.pallas_call(
        functools.partial(_kv_pack_body, bsz=B, heads=H),
        in_specs=[pl.BlockSpec(memory_space=pltpu.VMEM)] * 6,
        out_specs=[pl.BlockSpec(memory_space=pltpu.VMEM)] * 2,
        out_shape=[jax.ShapeDtypeStruct((B, d, d), BF16)] * 2,
    )(tm, wk_b, bk2, wv_b, bv2, mask_bd)

    hu, asum = pl.pallas_call(
        functools.partial(_attn_body, scale=(d // H) ** -0.5, heads=H),
        grid=(B, nT),
        in_specs=[
            pl.BlockSpec((1, TT, d), lambda b_, t_: (b_, t_, 0)),
            pl.BlockSpec((1, d, d), lambda b_, t_: (b_, 0, 0)),
            pl.BlockSpec((1, d, d), lambda b_, t_: (b_, 0, 0)),
            pl.BlockSpec((d, d), lambda b_, t_: (0, 0)),
            pl.BlockSpec((1, TT), lambda b_, t_: (0, 0)),
            pl.BlockSpec((d, d), lambda b_, t_: (0, 0)),
            pl.BlockSpec((1, d), lambda b_, t_: (0, 0)),
            pl.BlockSpec((d, d), lambda b_, t_: (0, 0)),
            pl.BlockSpec((1, d), lambda b_, t_: (0, 0)),
            pl.BlockSpec((1, d), lambda b_, t_: (0, 0)),
            pl.BlockSpec((1, d), lambda b_, t_: (0, 0)),
            pl.BlockSpec(memory_space=pltpu.SMEM),
        ],
        out_specs=[
            pl.BlockSpec((1, TT, d), lambda b_, t_: (b_, t_, 0)),
            pl.BlockSpec((1, 1, K), lambda b_, t_: (b_, 0, 0)),
        ],
        out_shape=[
            jax.ShapeDtypeStruct((B, T, d), F32),
            jax.ShapeDtypeStruct((B, 1, K), F32),
        ],
    )(h, km, vm, mask_bd, ones_tt, wq_b, bq2, wo_b, bo2, g2, b2, gl2)
    asum = asum.reshape(B, K)

    fa = pl.pallas_call(
        functools.partial(_scatter_body, inv_ht=1.0 / (H * T)),
        grid=(B,),
        in_specs=[
            pl.BlockSpec(memory_space=pltpu.SMEM),
            pl.BlockSpec(memory_space=pltpu.SMEM),
        ],
        out_specs=pl.BlockSpec((1, 1, N), lambda b_: (b_, 0, 0)),
        out_shape=jax.ShapeDtypeStruct((B, 1, N), F32),
    )(idx, asum)

    return hu, fa.reshape(B, N)
